# baseline (speedup 1.0000x reference)
import jax
import jax.numpy as jnp
from jax import lax
from jax.experimental import pallas as pl
from jax.experimental.pallas import tpu as pltpu
from jax.experimental.pallas import tpu_sc as plsc

BATCH = 4
SEQ = 2048
DIM = 64
VOCAB = 1000000
NUM_CORES = 2
NUM_SUBCORES = 16
NUM_WORKERS = NUM_CORES * NUM_SUBCORES
N = BATCH * SEQ
PER_W = N // NUM_WORKERS
GROUP = 16
N_GROUPS = PER_W // GROUP
NBUF = 4
LANES = 16


def _emb_body(x_hbm, tt_hbm, pe_hbm, out_hbm, idx_smem, idx_v, pe_v, blk, g_v, sems, pe_sem):
    wid = lax.axis_index("s") * NUM_CORES + lax.axis_index("c")
    base = wid * PER_W
    pe_base = base % SEQ

    pltpu.sync_copy(x_hbm.at[pl.ds(base, PER_W)], idx_v)
    pe_cp = pltpu.async_copy(pe_hbm.at[pl.ds(pe_base, PER_W)], pe_v, pe_sem)

    def spill(g, carry):
        vt = idx_v[pl.ds(g * GROUP, GROUP)]
        for j in range(GROUP):
            idx_smem[g * GROUP + j] = vt[j]
        return carry

    lax.fori_loop(0, N_GROUPS, spill, 0)
    pe_cp.wait()

    dvec = lax.iota(jnp.int32, LANES)

    def fire(t, slot):
        i = idx_smem[t]
        off = pl.multiple_of((i >> 7) * 128, 128)
        pltpu.async_copy(
            tt_hbm.at[:, pl.ds(off, 128)],
            blk.at[pl.ds(slot * DIM, DIM)],
            sems.at[slot],
        )

    def extract(t, slot):
        i = idx_smem[t]
        pltpu.make_async_copy(
            tt_hbm.at[:, pl.ds(0, 128)],
            blk.at[pl.ds(slot * DIM, DIM)],
            sems.at[slot],
        ).wait()
        cvec = jnp.zeros((LANES,), jnp.int32) + (i & 127)
        for k in range(DIM // LANES):
            vals = plsc.load_gather(blk, [slot * DIM + dvec + k * LANES, cvec])
            sl = pl.ds(k * LANES, LANES)
            g_v[t, sl] = vals + pe_v[t, sl]

    for s in range(NBUF):
        fire(s, s)

    def steady(b, carry):
        t0 = b * NBUF
        for s in range(NBUF):
            extract(t0 + s, s)

            @pl.when(t0 + s < PER_W - NBUF)
            def _():
                fire(t0 + s + NBUF, s)

        return carry

    lax.fori_loop(0, PER_W // NBUF, steady, 0)
    pltpu.sync_copy(g_v, out_hbm.at[pl.ds(base, PER_W)])


_emb_call = pl.kernel(
    _emb_body,
    out_type=jax.ShapeDtypeStruct((N, DIM), jnp.float32),
    mesh=plsc.VectorSubcoreMesh(
        core_axis_name="c",
        subcore_axis_name="s",
        num_cores=NUM_CORES,
        num_subcores=NUM_SUBCORES,
    ),
    compiler_params=pltpu.CompilerParams(needs_layout_passes=False),
    scratch_types=[
        pltpu.SMEM((PER_W,), jnp.int32),
        pltpu.VMEM((PER_W,), jnp.int32),
        pltpu.VMEM((PER_W, DIM), jnp.float32),
        pltpu.VMEM((NBUF * DIM, 128), jnp.float32),
        pltpu.VMEM((PER_W, DIM), jnp.float32),
        pltpu.SemaphoreType.DMA((NBUF,)),
        pltpu.SemaphoreType.DMA,
    ],
)


@jax.jit
def kernel(x, table, pe):
    out = _emb_call(x.reshape(N), table.T, pe)
    return out.reshape(BATCH, SEQ, DIM)


# speedup vs baseline: 1.0002x; 1.0002x over previous
import jax
import jax.numpy as jnp
from jax import lax
from jax.experimental import pallas as pl
from jax.experimental.pallas import tpu as pltpu
from jax.experimental.pallas import tpu_sc as plsc

BATCH = 4
SEQ = 2048
DIM = 64
VOCAB = 1000000
NUM_CORES = 2
NUM_SUBCORES = 16
NUM_WORKERS = NUM_CORES * NUM_SUBCORES
N = BATCH * SEQ
PER_W = N // NUM_WORKERS
GROUP = 16
N_GROUPS = PER_W // GROUP
NBUF = 4
LANES = 16


def _emb_body(x_hbm, tt_hbm, pe_hbm, out_hbm, idx_smem, idx_v, pe_v, blk, g_v, sems, pe_sem):
    wid = lax.axis_index("s") * NUM_CORES + lax.axis_index("c")
    base = wid * PER_W
    pe_base = base % SEQ

    pltpu.sync_copy(x_hbm.at[pl.ds(base, PER_W)], idx_v)
    pe_cp = pltpu.async_copy(pe_hbm.at[pl.ds(pe_base, PER_W)], pe_v, pe_sem)

    def spill(g, carry):
        vt = idx_v[pl.ds(g * GROUP, GROUP)]
        for j in range(GROUP):
            idx_smem[g * GROUP + j] = vt[j]
        return carry

    lax.fori_loop(0, N_GROUPS, spill, 0)
    pe_cp.wait()

    dvec = lax.iota(jnp.int32, LANES)

    def fire(t, slot):
        i = idx_smem[t]
        off = pl.multiple_of((i >> 7) * 128, 128)
        for a in range(DIM // 8):
            pltpu.async_copy(
                tt_hbm.at[pl.ds(a * 8, 8), pl.ds(off, 128)],
                blk.at[pl.ds(slot * DIM + a * 8, 8)],
                sems.at[slot],
            )

    def extract(t, slot):
        i = idx_smem[t]
        pltpu.make_async_copy(
            tt_hbm.at[:, pl.ds(0, 128)],
            blk.at[pl.ds(slot * DIM, DIM)],
            sems.at[slot],
        ).wait()
        cvec = jnp.zeros((LANES,), jnp.int32) + (i & 127)
        for k in range(DIM // LANES):
            vals = plsc.load_gather(blk, [slot * DIM + dvec + k * LANES, cvec])
            sl = pl.ds(k * LANES, LANES)
            g_v[t, sl] = vals + pe_v[t, sl]

    for s in range(NBUF):
        fire(s, s)

    def steady(b, carry):
        t0 = b * NBUF
        for s in range(NBUF):
            extract(t0 + s, s)

            @pl.when(t0 + s < PER_W - NBUF)
            def _():
                fire(t0 + s + NBUF, s)

        return carry

    lax.fori_loop(0, PER_W // NBUF, steady, 0)
    pltpu.sync_copy(g_v, out_hbm.at[pl.ds(base, PER_W)])


_emb_call = pl.kernel(
    _emb_body,
    out_type=jax.ShapeDtypeStruct((N, DIM), jnp.float32),
    mesh=plsc.VectorSubcoreMesh(
        core_axis_name="c",
        subcore_axis_name="s",
        num_cores=NUM_CORES,
        num_subcores=NUM_SUBCORES,
    ),
    compiler_params=pltpu.CompilerParams(needs_layout_passes=False),
    scratch_types=[
        pltpu.SMEM((PER_W,), jnp.int32),
        pltpu.VMEM((PER_W,), jnp.int32),
        pltpu.VMEM((PER_W, DIM), jnp.float32),
        pltpu.VMEM((NBUF * DIM, 128), jnp.float32),
        pltpu.VMEM((PER_W, DIM), jnp.float32),
        pltpu.SemaphoreType.DMA((NBUF,)),
        pltpu.SemaphoreType.DMA,
    ],
)


@jax.jit
def kernel(x, table, pe):
    out = _emb_call(x.reshape(N), table.T, pe)
    return out.reshape(BATCH, SEQ, DIM)


# speedup vs baseline: 1.1587x; 1.1584x over previous
import jax
import jax.numpy as jnp
from jax import lax
from jax.experimental import pallas as pl
from jax.experimental.pallas import tpu as pltpu
from jax.experimental.pallas import tpu_sc as plsc

BATCH = 4
SEQ = 2048
DIM = 64
VOCAB = 1000000
NUM_CORES = 2
NUM_SUBCORES = 16
NUM_WORKERS = NUM_CORES * NUM_SUBCORES
N = BATCH * SEQ
PER_W = N // NUM_WORKERS
GROUP = 16
N_GROUPS = PER_W // GROUP
NBUF = 8
LANES = 16


def _emb_body(x_hbm, tt_hbm, pe_hbm, out_hbm, idx_smem, idx_v, pe_v, blk, g_v, sems, pe_sem):
    wid = lax.axis_index("s") * NUM_CORES + lax.axis_index("c")
    base = wid * PER_W
    pe_base = base % SEQ

    pltpu.sync_copy(x_hbm.at[pl.ds(base, PER_W)], idx_v)
    pe_cp = pltpu.async_copy(pe_hbm.at[pl.ds(pe_base, PER_W)], pe_v, pe_sem)

    def spill(g, carry):
        vt = idx_v[pl.ds(g * GROUP, GROUP)]
        for j in range(GROUP):
            idx_smem[g * GROUP + j] = vt[j]
        return carry

    lax.fori_loop(0, N_GROUPS, spill, 0)
    pe_cp.wait()

    dvec = lax.iota(jnp.int32, LANES)

    def fire(t, slot):
        i = idx_smem[t]
        off = pl.multiple_of((i >> 7) * 128, 128)
        for a in range(DIM // 8):
            pltpu.async_copy(
                tt_hbm.at[pl.ds(a * 8, 8), pl.ds(off, 128)],
                blk.at[pl.ds(slot * DIM + a * 8, 8)],
                sems.at[slot],
            )

    def extract(t, slot):
        i = idx_smem[t]
        pltpu.make_async_copy(
            tt_hbm.at[:, pl.ds(0, 128)],
            blk.at[pl.ds(slot * DIM, DIM)],
            sems.at[slot],
        ).wait()
        cvec = jnp.zeros((LANES,), jnp.int32) + (i & 127)
        for k in range(DIM // LANES):
            vals = plsc.load_gather(blk, [slot * DIM + dvec + k * LANES, cvec])
            sl = pl.ds(k * LANES, LANES)
            g_v[t & (PER_W // 2 - 1), sl] = vals + pe_v[t, sl]

    for s in range(NBUF):
        fire(s, s)

    def steady(b, carry):
        t0 = b * NBUF
        for s in range(NBUF):
            extract(t0 + s, s)

            @pl.when(t0 + s < PER_W - NBUF)
            def _():
                fire(t0 + s + NBUF, s)

        @pl.when(b == PER_W // 2 // NBUF - 1)
        def _():
            pltpu.sync_copy(g_v, out_hbm.at[pl.ds(base, PER_W // 2)])

        return carry

    lax.fori_loop(0, PER_W // NBUF, steady, 0)
    pltpu.sync_copy(g_v, out_hbm.at[pl.ds(base + PER_W // 2, PER_W // 2)])


_emb_call = pl.kernel(
    _emb_body,
    out_type=jax.ShapeDtypeStruct((N, DIM), jnp.float32),
    mesh=plsc.VectorSubcoreMesh(
        core_axis_name="c",
        subcore_axis_name="s",
        num_cores=NUM_CORES,
        num_subcores=NUM_SUBCORES,
    ),
    compiler_params=pltpu.CompilerParams(needs_layout_passes=False),
    scratch_types=[
        pltpu.SMEM((PER_W,), jnp.int32),
        pltpu.VMEM((PER_W,), jnp.int32),
        pltpu.VMEM((PER_W, DIM), jnp.float32),
        pltpu.VMEM((NBUF * DIM, 128), jnp.float32),
        pltpu.VMEM((PER_W // 2, DIM), jnp.float32),
        pltpu.SemaphoreType.DMA((NBUF,)),
        pltpu.SemaphoreType.DMA,
    ],
)


@jax.jit
def kernel(x, table, pe):
    out = _emb_call(x.reshape(N), table.T, pe)
    return out.reshape(BATCH, SEQ, DIM)


# speedup vs baseline: 1.1628x; 1.0036x over previous
import jax
import jax.numpy as jnp
from jax import lax
from jax.experimental import pallas as pl
from jax.experimental.pallas import tpu as pltpu
from jax.experimental.pallas import tpu_sc as plsc

BATCH = 4
SEQ = 2048
DIM = 64
VOCAB = 1000000
NUM_CORES = 2
NUM_SUBCORES = 16
NUM_WORKERS = NUM_CORES * NUM_SUBCORES
N = BATCH * SEQ
PER_W = N // NUM_WORKERS
GROUP = 16
N_GROUPS = PER_W // GROUP
NBUF = 8
LANES = 16


def _emb_body(x_hbm, tt_hbm, pe_hbm, out_hbm, idx_smem, idx_v, pe_v, blk, g_v, sems, pe_sem):
    wid = lax.axis_index("s") * NUM_CORES + lax.axis_index("c")
    base = wid * PER_W
    pe_base = base % SEQ

    pltpu.sync_copy(x_hbm.at[pl.ds(base, PER_W)], idx_v)
    pe_cp = pltpu.async_copy(pe_hbm.at[pl.ds(pe_base, PER_W)], pe_v, pe_sem)

    def spill(g, carry):
        vt = idx_v[pl.ds(g * GROUP, GROUP)]
        for j in range(GROUP):
            idx_smem[g * GROUP + j] = vt[j]
        return carry

    spill(0, 0)

    dvec = lax.iota(jnp.int32, LANES)

    def fire(t, slot):
        i = idx_smem[t]
        off = pl.multiple_of((i >> 7) * 128, 128)
        pltpu.async_copy(
            tt_hbm.at[:, pl.ds(off, 128)],
            blk.at[pl.ds(slot * DIM, DIM)],
            sems.at[slot],
        )

    def extract(t, slot):
        i = idx_smem[t]
        pltpu.make_async_copy(
            tt_hbm.at[:, pl.ds(0, 128)],
            blk.at[pl.ds(slot * DIM, DIM)],
            sems.at[slot],
        ).wait()
        cvec = jnp.zeros((LANES,), jnp.int32) + (i & 127)
        for k in range(DIM // LANES):
            vals = plsc.load_gather(blk, [slot * DIM + dvec + k * LANES, cvec])
            sl = pl.ds(k * LANES, LANES)
            g_v[t & (PER_W // 2 - 1), sl] = vals + pe_v[t, sl]

    for s in range(NBUF):
        fire(s, s)

    lax.fori_loop(1, N_GROUPS, spill, 0)
    pe_cp.wait()

    def steady(b, carry):
        t0 = b * NBUF
        for s in range(NBUF):
            extract(t0 + s, s)

            @pl.when(t0 + s < PER_W - NBUF)
            def _():
                fire(t0 + s + NBUF, s)

        @pl.when(b == PER_W // 2 // NBUF - 1)
        def _():
            pltpu.sync_copy(g_v, out_hbm.at[pl.ds(base, PER_W // 2)])

        return carry

    lax.fori_loop(0, PER_W // NBUF, steady, 0)
    pltpu.sync_copy(g_v, out_hbm.at[pl.ds(base + PER_W // 2, PER_W // 2)])


_emb_call = pl.kernel(
    _emb_body,
    out_type=jax.ShapeDtypeStruct((N, DIM), jnp.float32),
    mesh=plsc.VectorSubcoreMesh(
        core_axis_name="c",
        subcore_axis_name="s",
        num_cores=NUM_CORES,
        num_subcores=NUM_SUBCORES,
    ),
    compiler_params=pltpu.CompilerParams(needs_layout_passes=False),
    scratch_types=[
        pltpu.SMEM((PER_W,), jnp.int32),
        pltpu.VMEM((PER_W,), jnp.int32),
        pltpu.VMEM((PER_W, DIM), jnp.float32),
        pltpu.VMEM((NBUF * DIM, 128), jnp.float32),
        pltpu.VMEM((PER_W // 2, DIM), jnp.float32),
        pltpu.SemaphoreType.DMA((NBUF,)),
        pltpu.SemaphoreType.DMA,
    ],
)


@jax.jit
def kernel(x, table, pe):
    out = _emb_call(x.reshape(N), table.T, pe)
    return out.reshape(BATCH, SEQ, DIM)
